# Initial kernel scaffold; baseline (speedup 1.0000x reference)
import jax
import jax.numpy as jnp
from jax.experimental import pallas as pl


def kernel(x, edge_index, batch, coeffs_t, node_W, node_b, g1_W1, g1_b1, g1_W2, g1_b2, g1_eps, g2_W1, g2_b1, g2_W2, g2_b2, g2_eps, g3_W1, g3_b1, g3_W2, g3_b2, g3_eps, fc1_W, fc1_b, fc2_W, fc2_b):
    raise NotImplementedError("write your pallas kernel here")


# speedup vs baseline: 7.2359x; 7.2359x over previous
import functools

import jax
import jax.numpy as jnp
from jax import lax
from jax.experimental import pallas as pl
from jax.experimental.pallas import tpu as pltpu
from jax.experimental.pallas import tpu_sc as plsc

_N = 50000
_F = 128
_H = 64
_HH = 32
_NG = 256
_NCLS = 10
_E = 800000

_SUB = 16
_CH = 125
_GRP = 16
_NCHUNK = _E // _CH
_CPS = _NCHUNK // _SUB
_GPS = _CPS // _GRP
_ZROWS = 400
_ZBLK = _N // _ZROWS

_BR = 2000
_NBLK = _N // _BR


def _bf(v):
    return v.astype(jnp.bfloat16).astype(jnp.float32)


def _interp3(cr, w_ref):
    return (_bf(cr[0]) * _bf(w_ref[0]) + _bf(cr[1]) * _bf(w_ref[1])
            + _bf(cr[2]) * _bf(w_ref[2]))


def _mm(a, b):
    return lax.dot_general(a.astype(jnp.bfloat16), b.astype(jnp.bfloat16),
                           (((1,), (1,)), ((), ())),
                           preferred_element_type=jnp.float32)


def _node_body(cr, x_ref, w_ref, b_ref, out_ref):
    wt = _interp3(cr, w_ref)
    bt = _interp3(cr, b_ref)
    h = _mm(x_ref[...], wt) + bt[None, :]
    out_ref[0] = h[:, :_HH]
    out_ref[1] = h[:, _HH:]


def _node_lin(x, node_w, node_b, coeffs):
    return pl.pallas_call(
        _node_body,
        grid=(_NBLK,),
        in_specs=[
            pl.BlockSpec(memory_space=pltpu.SMEM),
            pl.BlockSpec((_BR, _F), lambda r: (r, 0)),
            pl.BlockSpec((3, _H, _F), lambda r: (0, 0, 0)),
            pl.BlockSpec((3, _H), lambda r: (0, 0)),
        ],
        out_specs=pl.BlockSpec((2, _BR, _HH), lambda r: (0, r, 0)),
        out_shape=jax.ShapeDtypeStruct((2, _N, _HH), jnp.float32),
    )(coeffs, x, node_w, node_b)


def _gin_common(cr, er, h_ref, a_ref, w1_ref, b1_ref, w2_ref, b2_ref):
    eps_t = (_bf(cr[0]) * _bf(er[0]) + _bf(cr[1]) * _bf(er[1])
             + _bf(cr[2]) * _bf(er[2]))
    w1t = _interp3(cr, w1_ref)
    b1t = _interp3(cr, b1_ref)
    w2t = _interp3(cr, w2_ref)
    b2t = _interp3(cr, b2_ref)
    h = jnp.concatenate([h_ref[0], h_ref[1]], axis=1)
    agg = jnp.concatenate([a_ref[0], a_ref[1]], axis=1)
    u = (1.0 + eps_t) * h + agg
    m = jnp.maximum(_mm(u, w1t) + b1t[None, :], 0.0)
    o = _mm(m, w2t) + b2t[None, :]
    return jnp.where(o > 0.0, o, jnp.exp(o) - 1.0)


def _gin_body(cr, er, h_ref, a_ref, w1_ref, b1_ref, w2_ref, b2_ref, out_ref):
    o = _gin_common(cr, er, h_ref, a_ref, w1_ref, b1_ref, w2_ref, b2_ref)
    out_ref[0] = o[:, :_HH]
    out_ref[1] = o[:, _HH:]


def _gin_dense(hp, ap, w1, b1, w2, b2, eps, coeffs):
    return pl.pallas_call(
        _gin_body,
        grid=(_NBLK,),
        in_specs=[
            pl.BlockSpec(memory_space=pltpu.SMEM),
            pl.BlockSpec(memory_space=pltpu.SMEM),
            pl.BlockSpec((2, _BR, _HH), lambda r: (0, r, 0)),
            pl.BlockSpec((2, _BR, _HH), lambda r: (0, r, 0)),
            pl.BlockSpec((3, 2 * _H, _H), lambda r: (0, 0, 0)),
            pl.BlockSpec((3, 2 * _H), lambda r: (0, 0)),
            pl.BlockSpec((3, _H, 2 * _H), lambda r: (0, 0, 0)),
            pl.BlockSpec((3, _H), lambda r: (0, 0)),
        ],
        out_specs=pl.BlockSpec((2, _BR, _HH), lambda r: (0, r, 0)),
        out_shape=jax.ShapeDtypeStruct((2, _N, _HH), jnp.float32),
    )(coeffs, eps, hp, ap, w1, b1, w2, b2)


def _gin3_body(cr, er, h_ref, a_ref, w1_ref, b1_ref, w2_ref, b2_ref,
               idr_ref, idc_ref, fc1w_ref, fc1b_ref, fc2w_ref, fc2b_ref,
               psum_ref, pmax_ref, out_ref):
    r = pl.program_id(0)
    o = _gin_common(cr, er, h_ref, a_ref, w1_ref, b1_ref, w2_ref, b2_ref)
    ids_row = idr_ref[0]
    ids_col = idc_ref[0]

    iota_g = lax.broadcasted_iota(jnp.int32, (_NG, _BR), 0)
    oh = (ids_row == iota_g).astype(jnp.float32)
    o_aug = jnp.concatenate([o, jnp.ones((_BR, _H), jnp.float32)], axis=1)
    psum_blk = lax.dot_general(oh, o_aug, (((1,), (0,)), ((), ())),
                               precision=lax.Precision.HIGHEST,
                               preferred_element_type=jnp.float32)

    g_lo = ids_row[0, 0]
    g_hi = ids_row[0, _BR - 1]
    iota_n = lax.broadcasted_iota(jnp.int32, (_NG, 1), 0)
    neg_inf = jnp.float32(-jnp.inf)

    def wbody(carry):
        g, acc = carry
        mx = jnp.max(jnp.where(ids_col == g, o, neg_inf), axis=0)
        acc = jnp.where(iota_n == g, jnp.maximum(acc, mx[None, :]), acc)
        return g + 1, acc

    _, pmax_blk = lax.while_loop(lambda c: c[0] <= g_hi, wbody,
                                 (g_lo, jnp.full((_NG, _H), neg_inf)))

    @pl.when(r == 0)
    def _():
        psum_ref[...] = psum_blk
        pmax_ref[...] = pmax_blk

    @pl.when(r > 0)
    def _():
        psum_ref[...] = psum_ref[...] + psum_blk
        pmax_ref[...] = jnp.maximum(pmax_ref[...], pmax_blk)

    @pl.when(r == _NBLK - 1)
    def _():
        ps = psum_ref[...]
        mean = ps[:, :_H] / jnp.maximum(ps[:, _H:], 1.0)
        mx = pmax_ref[...]
        mx = jnp.where(jnp.isfinite(mx), mx, 0.0)
        z = jnp.concatenate([mean, mx], axis=1)
        fc1t = _interp3(cr, fc1w_ref)
        fc1bt = _interp3(cr, fc1b_ref)
        fc2t = _interp3(cr, fc2w_ref)
        fc2bt = _interp3(cr, fc2b_ref)
        z1 = jnp.maximum(_mm(z, fc1t) + fc1bt[None, :], 0.0)
        out_ref[...] = _mm(z1, fc2t) + fc2bt[None, :]


def _gin3_pool_head(hp, ap, w1, b1, w2, b2, eps, coeffs, ids_row, ids_col,
                    fc1w, fc1b, fc2w, fc2b):
    outs = pl.pallas_call(
        _gin3_body,
        grid=(_NBLK,),
        in_specs=[
            pl.BlockSpec(memory_space=pltpu.SMEM),
            pl.BlockSpec(memory_space=pltpu.SMEM),
            pl.BlockSpec((2, _BR, _HH), lambda r: (0, r, 0)),
            pl.BlockSpec((2, _BR, _HH), lambda r: (0, r, 0)),
            pl.BlockSpec((3, 2 * _H, _H), lambda r: (0, 0, 0)),
            pl.BlockSpec((3, 2 * _H), lambda r: (0, 0)),
            pl.BlockSpec((3, _H, 2 * _H), lambda r: (0, 0, 0)),
            pl.BlockSpec((3, _H), lambda r: (0, 0)),
            pl.BlockSpec((1, 1, _BR), lambda r: (r, 0, 0)),
            pl.BlockSpec((1, _BR, 1), lambda r: (r, 0, 0)),
            pl.BlockSpec((3, _H, 2 * _H), lambda r: (0, 0, 0)),
            pl.BlockSpec((3, _H), lambda r: (0, 0)),
            pl.BlockSpec((3, _NCLS, _H), lambda r: (0, 0, 0)),
            pl.BlockSpec((3, _NCLS), lambda r: (0, 0)),
        ],
        out_specs=[
            pl.BlockSpec((_NG, 2 * _H), lambda r: (0, 0)),
            pl.BlockSpec((_NG, _H), lambda r: (0, 0)),
            pl.BlockSpec((_NG, _NCLS), lambda r: (0, 0)),
        ],
        out_shape=[
            jax.ShapeDtypeStruct((_NG, 2 * _H), jnp.float32),
            jax.ShapeDtypeStruct((_NG, _H), jnp.float32),
            jax.ShapeDtypeStruct((_NG, _NCLS), jnp.float32),
        ],
    )(coeffs, eps, hp, ap, w1, b1, w2, b2, ids_row, ids_col,
      fc1w, fc1b, fc2w, fc2b)
    return outs[2]


def _edge_body(hflat, srcb, dstb, out, sidx, didx, rows0, rows1, zbuf,
               accum, gsem0, gsem1):
    c = lax.axis_index("c")
    s = lax.axis_index("s")
    c_n = c * _N

    zero16 = jnp.zeros((16,), jnp.float32)

    def zb(j, carry):
        zbuf[j, pl.ds(0, 16)] = zero16
        zbuf[j, pl.ds(16, 16)] = zero16
        return carry

    lax.fori_loop(0, _ZROWS, zb, 0)
    for k in range(8):
        t = s + _SUB * k

        @pl.when(t < _ZBLK)
        def _():
            pltpu.sync_copy(zbuf, accum.at[pl.ds(t * _ZROWS, _ZROWS)])

    plsc.subcore_barrier()

    base_chunk = s * _CPS

    def group(g, carry):
        g0 = base_chunk + g * _GRP
        pltpu.sync_copy(srcb.at[pl.ds(c * _NCHUNK + g0, _GRP)], sidx)
        pltpu.sync_copy(dstb.at[pl.ds(g0, _GRP)], didx)
        desc = pltpu.async_copy(hflat.at[sidx.at[0, 0]], rows0, gsem0)
        for j in range(_GRP):
            rbuf = rows0 if j % 2 == 0 else rows1
            nbuf = rows1 if j % 2 == 0 else rows0
            nsem = gsem1 if j % 2 == 0 else gsem0
            if j + 1 < _GRP:
                nxt = pltpu.async_copy(hflat.at[sidx.at[j + 1, 0]], nbuf, nsem)
            desc.wait()
            pltpu.sync_copy(rbuf, accum.at[didx.at[j, 0]], add=True)
            if j + 1 < _GRP:
                desc = nxt
        return carry

    lax.fori_loop(0, _GPS, group, 0)

    plsc.subcore_barrier()
    for k in range(8):
        t = s + _SUB * k

        @pl.when(t < _ZBLK)
        def _():
            pltpu.sync_copy(accum.at[pl.ds(t * _ZROWS, _ZROWS)],
                            out.at[pl.ds(c_n + t * _ZROWS, _ZROWS)])


@functools.lru_cache(maxsize=1)
def _make_edge_agg():
    return pl.kernel(
        _edge_body,
        out_type=jax.ShapeDtypeStruct((2 * _N, _HH), jnp.float32),
        mesh=plsc.VectorSubcoreMesh(core_axis_name="c", subcore_axis_name="s"),
        compiler_params=pltpu.CompilerParams(use_tc_tiling_on_sc=False),
        scratch_types=[
            pltpu.VMEM((_GRP, 1, _CH), jnp.int32),
            pltpu.VMEM((_GRP, 1, _CH), jnp.int32),
            pltpu.VMEM((_CH, _HH), jnp.float32),
            pltpu.VMEM((_CH, _HH), jnp.float32),
            pltpu.VMEM((_ZROWS, _HH), jnp.float32),
            pltpu.VMEM_SHARED((_N, _HH), jnp.float32),
            pltpu.SemaphoreType.DMA,
            pltpu.SemaphoreType.DMA,
        ],
    )


def _edge_agg(hflat, srcb, dstb):
    return _make_edge_agg()(hflat, srcb, dstb)


def kernel(x, edge_index, batch, coeffs_t, node_W, node_b,
           g1_W1, g1_b1, g1_W2, g1_b2, g1_eps,
           g2_W1, g2_b1, g2_W2, g2_b2, g2_eps,
           g3_W1, g3_b1, g3_W2, g3_b2, g3_eps,
           fc1_W, fc1_b, fc2_W, fc2_b):
    src = edge_index[0].reshape(_NCHUNK, 1, _CH)
    dst = edge_index[1].reshape(_NCHUNK, 1, _CH)
    srcb = jnp.concatenate([src, src + _N], axis=0)
    ids_row = batch.reshape(_NBLK, 1, _BR)
    ids_col = batch.reshape(_NBLK, _BR, 1)

    hp = _node_lin(x, node_W, node_b, coeffs_t)
    agg = _edge_agg(hp.reshape(2 * _N, _HH), srcb, dst)
    hp = _gin_dense(hp, agg.reshape(2, _N, _HH),
                    g1_W1, g1_b1, g1_W2, g1_b2, g1_eps, coeffs_t)
    agg = _edge_agg(hp.reshape(2 * _N, _HH), srcb, dst)
    hp = _gin_dense(hp, agg.reshape(2, _N, _HH),
                    g2_W1, g2_b1, g2_W2, g2_b2, g2_eps, coeffs_t)
    agg = _edge_agg(hp.reshape(2 * _N, _HH), srcb, dst)
    return _gin3_pool_head(hp, agg.reshape(2, _N, _HH),
                           g3_W1, g3_b1, g3_W2, g3_b2, g3_eps, coeffs_t,
                           ids_row, ids_col, fc1_W, fc1_b, fc2_W, fc2_b)


# speedup vs baseline: 8.7516x; 1.2095x over previous
import functools

import jax
import jax.numpy as jnp
from jax import lax
from jax.experimental import pallas as pl
from jax.experimental.pallas import tpu as pltpu
from jax.experimental.pallas import tpu_sc as plsc

_N = 50000
_F = 128
_H = 64
_HH = 32
_NG = 256
_NCLS = 10
_E = 800000

_SUB = 16
_CH = 125
_GRP = 10
_NCHUNK = _E // _CH
_CPS = _NCHUNK // _SUB
_GPS = _CPS // _GRP
_GPAIR = _GPS // 2
_ZROWS = 200
_ZBLK = _N // _ZROWS

_BR = 2000
_NBLK = _N // _BR


def _bf(v):
    return v.astype(jnp.bfloat16).astype(jnp.float32)


def _interp3(cr, w_ref):
    return (_bf(cr[0]) * _bf(w_ref[0]) + _bf(cr[1]) * _bf(w_ref[1])
            + _bf(cr[2]) * _bf(w_ref[2]))


def _mm(a, b):
    return lax.dot_general(a.astype(jnp.bfloat16), b.astype(jnp.bfloat16),
                           (((1,), (1,)), ((), ())),
                           preferred_element_type=jnp.float32)


def _node_body(cr, x_ref, w_ref, b_ref, out_ref):
    wt = _interp3(cr, w_ref)
    bt = _interp3(cr, b_ref)
    h = _mm(x_ref[...], wt) + bt[None, :]
    out_ref[0] = h[:, :_HH]
    out_ref[1] = h[:, _HH:]


def _node_lin(x, node_w, node_b, coeffs):
    return pl.pallas_call(
        _node_body,
        grid=(_NBLK,),
        in_specs=[
            pl.BlockSpec(memory_space=pltpu.SMEM),
            pl.BlockSpec((_BR, _F), lambda r: (r, 0)),
            pl.BlockSpec((3, _H, _F), lambda r: (0, 0, 0)),
            pl.BlockSpec((3, _H), lambda r: (0, 0)),
        ],
        out_specs=pl.BlockSpec((2, _BR, _HH), lambda r: (0, r, 0)),
        out_shape=jax.ShapeDtypeStruct((2, _N, _HH), jnp.float32),
    )(coeffs, x, node_w, node_b)


def _gin_common(cr, er, h_ref, a_ref, w1_ref, b1_ref, w2_ref, b2_ref):
    eps_t = (_bf(cr[0]) * _bf(er[0]) + _bf(cr[1]) * _bf(er[1])
             + _bf(cr[2]) * _bf(er[2]))
    w1t = _interp3(cr, w1_ref)
    b1t = _interp3(cr, b1_ref)
    w2t = _interp3(cr, w2_ref)
    b2t = _interp3(cr, b2_ref)
    h = jnp.concatenate([h_ref[0], h_ref[1]], axis=1)
    agg = jnp.concatenate([a_ref[0], a_ref[1]], axis=1)
    u = (1.0 + eps_t) * h + agg
    m = jnp.maximum(_mm(u, w1t) + b1t[None, :], 0.0)
    o = _mm(m, w2t) + b2t[None, :]
    return jnp.where(o > 0.0, o, jnp.exp(o) - 1.0)


def _gin_body(cr, er, h_ref, a_ref, w1_ref, b1_ref, w2_ref, b2_ref, out_ref):
    o = _gin_common(cr, er, h_ref, a_ref, w1_ref, b1_ref, w2_ref, b2_ref)
    out_ref[0] = o[:, :_HH]
    out_ref[1] = o[:, _HH:]


def _gin_dense(hp, ap, w1, b1, w2, b2, eps, coeffs):
    return pl.pallas_call(
        _gin_body,
        grid=(_NBLK,),
        in_specs=[
            pl.BlockSpec(memory_space=pltpu.SMEM),
            pl.BlockSpec(memory_space=pltpu.SMEM),
            pl.BlockSpec((2, _BR, _HH), lambda r: (0, r, 0)),
            pl.BlockSpec((2, _BR, _HH), lambda r: (0, r, 0)),
            pl.BlockSpec((3, 2 * _H, _H), lambda r: (0, 0, 0)),
            pl.BlockSpec((3, 2 * _H), lambda r: (0, 0)),
            pl.BlockSpec((3, _H, 2 * _H), lambda r: (0, 0, 0)),
            pl.BlockSpec((3, _H), lambda r: (0, 0)),
        ],
        out_specs=pl.BlockSpec((2, _BR, _HH), lambda r: (0, r, 0)),
        out_shape=jax.ShapeDtypeStruct((2, _N, _HH), jnp.float32),
    )(coeffs, eps, hp, ap, w1, b1, w2, b2)


def _gin3_body(cr, er, h_ref, a_ref, w1_ref, b1_ref, w2_ref, b2_ref,
               idr_ref, idc_ref, fc1w_ref, fc1b_ref, fc2w_ref, fc2b_ref,
               psum_ref, pmax_ref, out_ref):
    r = pl.program_id(0)
    o = _gin_common(cr, er, h_ref, a_ref, w1_ref, b1_ref, w2_ref, b2_ref)
    ids_row = idr_ref[0]
    ids_col = idc_ref[0]

    iota_g = lax.broadcasted_iota(jnp.int32, (_NG, _BR), 0)
    oh = (ids_row == iota_g).astype(jnp.float32)
    o_aug = jnp.concatenate([o, jnp.ones((_BR, _H), jnp.float32)], axis=1)
    psum_blk = lax.dot_general(oh, o_aug, (((1,), (0,)), ((), ())),
                               precision=lax.Precision.HIGHEST,
                               preferred_element_type=jnp.float32)

    g_lo = ids_row[0, 0]
    g_hi = ids_row[0, _BR - 1]
    iota_n = lax.broadcasted_iota(jnp.int32, (_NG, 1), 0)
    neg_inf = jnp.float32(-jnp.inf)

    def wbody(carry):
        g, acc = carry
        mx = jnp.max(jnp.where(ids_col == g, o, neg_inf), axis=0)
        acc = jnp.where(iota_n == g, jnp.maximum(acc, mx[None, :]), acc)
        return g + 1, acc

    _, pmax_blk = lax.while_loop(lambda c: c[0] <= g_hi, wbody,
                                 (g_lo, jnp.full((_NG, _H), neg_inf)))

    @pl.when(r == 0)
    def _():
        psum_ref[...] = psum_blk
        pmax_ref[...] = pmax_blk

    @pl.when(r > 0)
    def _():
        psum_ref[...] = psum_ref[...] + psum_blk
        pmax_ref[...] = jnp.maximum(pmax_ref[...], pmax_blk)

    @pl.when(r == _NBLK - 1)
    def _():
        ps = psum_ref[...]
        mean = ps[:, :_H] / jnp.maximum(ps[:, _H:], 1.0)
        mx = pmax_ref[...]
        mx = jnp.where(jnp.isfinite(mx), mx, 0.0)
        z = jnp.concatenate([mean, mx], axis=1)
        fc1t = _interp3(cr, fc1w_ref)
        fc1bt = _interp3(cr, fc1b_ref)
        fc2t = _interp3(cr, fc2w_ref)
        fc2bt = _interp3(cr, fc2b_ref)
        z1 = jnp.maximum(_mm(z, fc1t) + fc1bt[None, :], 0.0)
        out_ref[...] = _mm(z1, fc2t) + fc2bt[None, :]


def _gin3_pool_head(hp, ap, w1, b1, w2, b2, eps, coeffs, ids_row, ids_col,
                    fc1w, fc1b, fc2w, fc2b):
    outs = pl.pallas_call(
        _gin3_body,
        grid=(_NBLK,),
        in_specs=[
            pl.BlockSpec(memory_space=pltpu.SMEM),
            pl.BlockSpec(memory_space=pltpu.SMEM),
            pl.BlockSpec((2, _BR, _HH), lambda r: (0, r, 0)),
            pl.BlockSpec((2, _BR, _HH), lambda r: (0, r, 0)),
            pl.BlockSpec((3, 2 * _H, _H), lambda r: (0, 0, 0)),
            pl.BlockSpec((3, 2 * _H), lambda r: (0, 0)),
            pl.BlockSpec((3, _H, 2 * _H), lambda r: (0, 0, 0)),
            pl.BlockSpec((3, _H), lambda r: (0, 0)),
            pl.BlockSpec((1, 1, _BR), lambda r: (r, 0, 0)),
            pl.BlockSpec((1, _BR, 1), lambda r: (r, 0, 0)),
            pl.BlockSpec((3, _H, 2 * _H), lambda r: (0, 0, 0)),
            pl.BlockSpec((3, _H), lambda r: (0, 0)),
            pl.BlockSpec((3, _NCLS, _H), lambda r: (0, 0, 0)),
            pl.BlockSpec((3, _NCLS), lambda r: (0, 0)),
        ],
        out_specs=[
            pl.BlockSpec((_NG, 2 * _H), lambda r: (0, 0)),
            pl.BlockSpec((_NG, _H), lambda r: (0, 0)),
            pl.BlockSpec((_NG, _NCLS), lambda r: (0, 0)),
        ],
        out_shape=[
            jax.ShapeDtypeStruct((_NG, 2 * _H), jnp.float32),
            jax.ShapeDtypeStruct((_NG, _H), jnp.float32),
            jax.ShapeDtypeStruct((_NG, _NCLS), jnp.float32),
        ],
    )(coeffs, eps, hp, ap, w1, b1, w2, b2, ids_row, ids_col,
      fc1w, fc1b, fc2w, fc2b)
    return outs[2]


def _edge_body(hflat, srcb, dstb, out,
               sidx_a, didx_a, sidx_b, didx_b,
               rows0, rows1, rows2, rows3, zbuf, accum,
               gsem0, gsem1, gsem2, gsem3,
               ssem0, ssem1, ssem2, ssem3, isem_a, isem_b):
    c = lax.axis_index("c")
    s = lax.axis_index("s")
    c_n = c * _N
    rows = (rows0, rows1, rows2, rows3)
    gsems = (gsem0, gsem1, gsem2, gsem3)
    ssems = (ssem0, ssem1, ssem2, ssem3)
    base = s * _CPS

    def fire_idx(g, sidx, didx, isem):
        g0 = base + g * _GRP
        pltpu.async_copy(srcb.at[pl.ds(c * _NCHUNK + g0, _GRP)], sidx, isem)
        pltpu.async_copy(dstb.at[pl.ds(g0, _GRP)], didx, isem)

    def drain_idx(sidx, didx, isem):
        pltpu.make_async_copy(srcb.at[pl.ds(0, _GRP)], sidx, isem).wait()
        pltpu.make_async_copy(dstb.at[pl.ds(0, _GRP)], didx, isem).wait()

    fire_idx(0, sidx_a, didx_a, isem_a)

    zero16 = jnp.zeros((16,), jnp.float32)

    def zb(j, carry):
        zbuf[j, pl.ds(0, 16)] = zero16
        zbuf[j, pl.ds(16, 16)] = zero16
        return carry

    lax.fori_loop(0, _ZROWS, zb, 0)
    for k in range(16):
        t = s + _SUB * k

        @pl.when(t < _ZBLK)
        def _():
            pltpu.sync_copy(zbuf, accum.at[pl.ds(t * _ZROWS, _ZROWS)])

    plsc.subcore_barrier()

    def process(sidx, didx):
        gd = [None] * _GRP
        sd = [None] * _GRP
        gd[0] = pltpu.async_copy(hflat.at[sidx.at[0, 0]], rows[0], gsems[0])
        gd[1] = pltpu.async_copy(hflat.at[sidx.at[1, 0]], rows[1], gsems[1])
        for j in range(_GRP):
            b = j % 4
            if j + 2 < _GRP:
                nb = (j + 2) % 4
                if j >= 2:
                    sd[j - 2].wait()
                gd[j + 2] = pltpu.async_copy(hflat.at[sidx.at[j + 2, 0]],
                                             rows[nb], gsems[nb])
            gd[j].wait()
            sd[j] = pltpu.async_copy(rows[b], accum.at[didx.at[j, 0]],
                                     ssems[b], add=True)
        for j in range(_GRP - 4, _GRP):
            sd[j].wait()

    def pair(gg, carry):
        fire_idx(2 * gg + 1, sidx_b, didx_b, isem_b)
        drain_idx(sidx_a, didx_a, isem_a)
        process(sidx_a, didx_a)

        @pl.when(gg < _GPAIR - 1)
        def _():
            fire_idx(2 * gg + 2, sidx_a, didx_a, isem_a)

        drain_idx(sidx_b, didx_b, isem_b)
        process(sidx_b, didx_b)
        return carry

    lax.fori_loop(0, _GPAIR, pair, 0)

    plsc.subcore_barrier()
    for k in range(16):
        t = s + _SUB * k

        @pl.when(t < _ZBLK)
        def _():
            pltpu.sync_copy(accum.at[pl.ds(t * _ZROWS, _ZROWS)],
                            out.at[pl.ds(c_n + t * _ZROWS, _ZROWS)])


@functools.lru_cache(maxsize=1)
def _make_edge_agg():
    return pl.kernel(
        _edge_body,
        out_type=jax.ShapeDtypeStruct((2 * _N, _HH), jnp.float32),
        mesh=plsc.VectorSubcoreMesh(core_axis_name="c", subcore_axis_name="s"),
        compiler_params=pltpu.CompilerParams(use_tc_tiling_on_sc=False),
        scratch_types=[
            pltpu.VMEM((_GRP, 1, _CH), jnp.int32),
            pltpu.VMEM((_GRP, 1, _CH), jnp.int32),
            pltpu.VMEM((_GRP, 1, _CH), jnp.int32),
            pltpu.VMEM((_GRP, 1, _CH), jnp.int32),
            pltpu.VMEM((_CH, _HH), jnp.float32),
            pltpu.VMEM((_CH, _HH), jnp.float32),
            pltpu.VMEM((_CH, _HH), jnp.float32),
            pltpu.VMEM((_CH, _HH), jnp.float32),
            pltpu.VMEM((_ZROWS, _HH), jnp.float32),
            pltpu.VMEM_SHARED((_N, _HH), jnp.float32),
            pltpu.SemaphoreType.DMA,
            pltpu.SemaphoreType.DMA,
            pltpu.SemaphoreType.DMA,
            pltpu.SemaphoreType.DMA,
            pltpu.SemaphoreType.DMA,
            pltpu.SemaphoreType.DMA,
            pltpu.SemaphoreType.DMA,
            pltpu.SemaphoreType.DMA,
            pltpu.SemaphoreType.DMA,
            pltpu.SemaphoreType.DMA,
        ],
    )


def _edge_agg(hflat, srcb, dstb):
    return _make_edge_agg()(hflat, srcb, dstb)


def kernel(x, edge_index, batch, coeffs_t, node_W, node_b,
           g1_W1, g1_b1, g1_W2, g1_b2, g1_eps,
           g2_W1, g2_b1, g2_W2, g2_b2, g2_eps,
           g3_W1, g3_b1, g3_W2, g3_b2, g3_eps,
           fc1_W, fc1_b, fc2_W, fc2_b):
    src = edge_index[0].reshape(_NCHUNK, 1, _CH)
    dst = edge_index[1].reshape(_NCHUNK, 1, _CH)
    srcb = jnp.concatenate([src, src + _N], axis=0)
    ids_row = batch.reshape(_NBLK, 1, _BR)
    ids_col = batch.reshape(_NBLK, _BR, 1)

    hp = _node_lin(x, node_W, node_b, coeffs_t)
    agg = _edge_agg(hp.reshape(2 * _N, _HH), srcb, dst)
    hp = _gin_dense(hp, agg.reshape(2, _N, _HH),
                    g1_W1, g1_b1, g1_W2, g1_b2, g1_eps, coeffs_t)
    agg = _edge_agg(hp.reshape(2 * _N, _HH), srcb, dst)
    hp = _gin_dense(hp, agg.reshape(2, _N, _HH),
                    g2_W1, g2_b1, g2_W2, g2_b2, g2_eps, coeffs_t)
    agg = _edge_agg(hp.reshape(2 * _N, _HH), srcb, dst)
    return _gin3_pool_head(hp, agg.reshape(2, _N, _HH),
                           g3_W1, g3_b1, g3_W2, g3_b2, g3_eps, coeffs_t,
                           ids_row, ids_col, fc1_W, fc1_b, fc2_W, fc2_b)


# speedup vs baseline: 8.8836x; 1.0151x over previous
import functools

import jax
import jax.numpy as jnp
from jax import lax
from jax.experimental import pallas as pl
from jax.experimental.pallas import tpu as pltpu
from jax.experimental.pallas import tpu_sc as plsc

_N = 50000
_F = 128
_H = 64
_HH = 32
_NG = 256
_NCLS = 10
_E = 800000

_SUB = 16
_CH = 125
_GRP = 10
_NCHUNK = _E // _CH
_CPS = _NCHUNK // _SUB
_GPS = _CPS // _GRP
_GPAIR = _GPS // 2
_ZROWS = 200
_ZBLK = _N // _ZROWS

_BR = 2000
_NBLK = _N // _BR
_BR3 = 1000
_NBLK3 = _N // _BR3


def _bf(v):
    return v.astype(jnp.bfloat16).astype(jnp.float32)


def _interp3(cr, w_ref):
    return (_bf(cr[0]) * _bf(w_ref[0]) + _bf(cr[1]) * _bf(w_ref[1])
            + _bf(cr[2]) * _bf(w_ref[2]))


def _mm(a, b):
    return lax.dot_general(a.astype(jnp.bfloat16), b.astype(jnp.bfloat16),
                           (((1,), (1,)), ((), ())),
                           preferred_element_type=jnp.float32)


def _node_body(cr, x_ref, w_ref, b_ref, out_ref):
    wt = _interp3(cr, w_ref)
    bt = _interp3(cr, b_ref)
    h = _mm(x_ref[...], wt) + bt[None, :]
    out_ref[0] = h[:, :_HH]
    out_ref[1] = h[:, _HH:]


def _node_lin(x, node_w, node_b, coeffs):
    return pl.pallas_call(
        _node_body,
        grid=(_NBLK,),
        in_specs=[
            pl.BlockSpec(memory_space=pltpu.SMEM),
            pl.BlockSpec((_BR, _F), lambda r: (r, 0)),
            pl.BlockSpec((3, _H, _F), lambda r: (0, 0, 0)),
            pl.BlockSpec((3, _H), lambda r: (0, 0)),
        ],
        out_specs=pl.BlockSpec((2, _BR, _HH), lambda r: (0, r, 0)),
        out_shape=jax.ShapeDtypeStruct((2, _N, _HH), jnp.float32),
    )(coeffs, x, node_w, node_b)


def _gin_common(cr, er, h_ref, a_ref, w1_ref, b1_ref, w2_ref, b2_ref):
    eps_t = (_bf(cr[0]) * _bf(er[0]) + _bf(cr[1]) * _bf(er[1])
             + _bf(cr[2]) * _bf(er[2]))
    w1t = _interp3(cr, w1_ref)
    b1t = _interp3(cr, b1_ref)
    w2t = _interp3(cr, w2_ref)
    b2t = _interp3(cr, b2_ref)
    h = jnp.concatenate([h_ref[0], h_ref[1]], axis=1)
    agg = jnp.concatenate([a_ref[0], a_ref[1]], axis=1)
    u = (1.0 + eps_t) * h + agg
    m = jnp.maximum(_mm(u, w1t) + b1t[None, :], 0.0)
    o = _mm(m, w2t) + b2t[None, :]
    return jnp.where(o > 0.0, o, jnp.exp(o) - 1.0)


def _gin_body(cr, er, h_ref, a_ref, w1_ref, b1_ref, w2_ref, b2_ref, out_ref):
    o = _gin_common(cr, er, h_ref, a_ref, w1_ref, b1_ref, w2_ref, b2_ref)
    out_ref[0] = o[:, :_HH]
    out_ref[1] = o[:, _HH:]


def _gin_dense(hp, ap, w1, b1, w2, b2, eps, coeffs):
    return pl.pallas_call(
        _gin_body,
        grid=(_NBLK,),
        in_specs=[
            pl.BlockSpec(memory_space=pltpu.SMEM),
            pl.BlockSpec(memory_space=pltpu.SMEM),
            pl.BlockSpec((2, _BR, _HH), lambda r: (0, r, 0)),
            pl.BlockSpec((2, _BR, _HH), lambda r: (0, r, 0)),
            pl.BlockSpec((3, 2 * _H, _H), lambda r: (0, 0, 0)),
            pl.BlockSpec((3, 2 * _H), lambda r: (0, 0)),
            pl.BlockSpec((3, _H, 2 * _H), lambda r: (0, 0, 0)),
            pl.BlockSpec((3, _H), lambda r: (0, 0)),
        ],
        out_specs=pl.BlockSpec((2, _BR, _HH), lambda r: (0, r, 0)),
        out_shape=jax.ShapeDtypeStruct((2, _N, _HH), jnp.float32),
    )(coeffs, eps, hp, ap, w1, b1, w2, b2)


def _gin3_body(cr, er, h_ref, a_ref, w1_ref, b1_ref, w2_ref, b2_ref,
               idr_ref, idc_ref, fc1w_ref, fc1b_ref, fc2w_ref, fc2b_ref,
               psum_ref, pmax_ref, out_ref):
    r = pl.program_id(0)
    o = _gin_common(cr, er, h_ref, a_ref, w1_ref, b1_ref, w2_ref, b2_ref)
    ids_row = idr_ref[0]
    ids_col = idc_ref[0]

    iota_g = lax.broadcasted_iota(jnp.int32, (_NG, _BR3), 0)
    oh = (ids_row == iota_g).astype(jnp.float32)
    o_aug = jnp.concatenate([o, jnp.ones((_BR3, _H), jnp.float32)], axis=1)
    psum_blk = lax.dot_general(oh, o_aug, (((1,), (0,)), ((), ())),
                               precision=lax.Precision.HIGHEST,
                               preferred_element_type=jnp.float32)

    g_lo = ids_row[0, 0]
    g_hi = ids_row[0, _BR3 - 1]
    iota_n = lax.broadcasted_iota(jnp.int32, (_NG, 1), 0)
    neg_inf = jnp.float32(-jnp.inf)

    def wbody(carry):
        g, acc = carry
        mx = jnp.max(jnp.where(ids_col == g, o, neg_inf), axis=0)
        acc = jnp.where(iota_n == g, jnp.maximum(acc, mx[None, :]), acc)
        return g + 1, acc

    _, pmax_blk = lax.while_loop(lambda c: c[0] <= g_hi, wbody,
                                 (g_lo, jnp.full((_NG, _H), neg_inf)))

    @pl.when(r == 0)
    def _():
        psum_ref[...] = psum_blk
        pmax_ref[...] = pmax_blk

    @pl.when(r > 0)
    def _():
        psum_ref[...] = psum_ref[...] + psum_blk
        pmax_ref[...] = jnp.maximum(pmax_ref[...], pmax_blk)

    @pl.when(r == _NBLK3 - 1)
    def _():
        ps = psum_ref[...]
        mean = ps[:, :_H] / jnp.maximum(ps[:, _H:], 1.0)
        mx = pmax_ref[...]
        mx = jnp.where(jnp.isfinite(mx), mx, 0.0)
        z = jnp.concatenate([mean, mx], axis=1)
        fc1t = _interp3(cr, fc1w_ref)
        fc1bt = _interp3(cr, fc1b_ref)
        fc2t = _interp3(cr, fc2w_ref)
        fc2bt = _interp3(cr, fc2b_ref)
        z1 = jnp.maximum(_mm(z, fc1t) + fc1bt[None, :], 0.0)
        out_ref[...] = _mm(z1, fc2t) + fc2bt[None, :]


def _gin3_pool_head(hp, ap, w1, b1, w2, b2, eps, coeffs, ids_row, ids_col,
                    fc1w, fc1b, fc2w, fc2b):
    outs = pl.pallas_call(
        _gin3_body,
        grid=(_NBLK3,),
        in_specs=[
            pl.BlockSpec(memory_space=pltpu.SMEM),
            pl.BlockSpec(memory_space=pltpu.SMEM),
            pl.BlockSpec((2, _BR3, _HH), lambda r: (0, r, 0)),
            pl.BlockSpec((2, _BR3, _HH), lambda r: (0, r, 0)),
            pl.BlockSpec((3, 2 * _H, _H), lambda r: (0, 0, 0)),
            pl.BlockSpec((3, 2 * _H), lambda r: (0, 0)),
            pl.BlockSpec((3, _H, 2 * _H), lambda r: (0, 0, 0)),
            pl.BlockSpec((3, _H), lambda r: (0, 0)),
            pl.BlockSpec((1, 1, _BR3), lambda r: (r, 0, 0)),
            pl.BlockSpec((1, _BR3, 1), lambda r: (r, 0, 0)),
            pl.BlockSpec((3, _H, 2 * _H), lambda r: (0, 0, 0)),
            pl.BlockSpec((3, _H), lambda r: (0, 0)),
            pl.BlockSpec((3, _NCLS, _H), lambda r: (0, 0, 0)),
            pl.BlockSpec((3, _NCLS), lambda r: (0, 0)),
        ],
        out_specs=[
            pl.BlockSpec((_NG, 2 * _H), lambda r: (0, 0)),
            pl.BlockSpec((_NG, _H), lambda r: (0, 0)),
            pl.BlockSpec((_NG, _NCLS), lambda r: (0, 0)),
        ],
        out_shape=[
            jax.ShapeDtypeStruct((_NG, 2 * _H), jnp.float32),
            jax.ShapeDtypeStruct((_NG, _H), jnp.float32),
            jax.ShapeDtypeStruct((_NG, _NCLS), jnp.float32),
        ],
    )(coeffs, eps, hp, ap, w1, b1, w2, b2, ids_row, ids_col,
      fc1w, fc1b, fc2w, fc2b)
    return outs[2]


def _edge_body(hflat, srcb, dstb, out,
               sidx_a, didx_a, sidx_b, didx_b,
               rows0, rows1, rows2, rows3, zbuf, accum,
               gsem0, gsem1, gsem2, gsem3,
               ssem0, ssem1, ssem2, ssem3, isem_a, isem_b):
    c = lax.axis_index("c")
    s = lax.axis_index("s")
    c_n = c * _N
    rows = (rows0, rows1, rows2, rows3)
    gsems = (gsem0, gsem1, gsem2, gsem3)
    ssems = (ssem0, ssem1, ssem2, ssem3)
    base = s * _CPS

    def fire_idx(g, sidx, didx, isem):
        g0 = base + g * _GRP
        pltpu.async_copy(srcb.at[pl.ds(c * _NCHUNK + g0, _GRP)], sidx, isem)
        pltpu.async_copy(dstb.at[pl.ds(g0, _GRP)], didx, isem)

    def drain_idx(sidx, didx, isem):
        pltpu.make_async_copy(srcb.at[pl.ds(0, _GRP)], sidx, isem).wait()
        pltpu.make_async_copy(dstb.at[pl.ds(0, _GRP)], didx, isem).wait()

    fire_idx(0, sidx_a, didx_a, isem_a)

    zero16 = jnp.zeros((16,), jnp.float32)

    def zb(j, carry):
        zbuf[j, pl.ds(0, 16)] = zero16
        zbuf[j, pl.ds(16, 16)] = zero16
        return carry

    lax.fori_loop(0, _ZROWS, zb, 0)
    for k in range(16):
        t = s + _SUB * k

        @pl.when(t < _ZBLK)
        def _():
            pltpu.sync_copy(zbuf, accum.at[pl.ds(t * _ZROWS, _ZROWS)])

    plsc.subcore_barrier()

    def process(sidx, didx):
        gd = [None] * _GRP
        sd = [None] * _GRP
        gd[0] = pltpu.async_copy(hflat.at[sidx.at[0, 0]], rows[0], gsems[0])
        gd[1] = pltpu.async_copy(hflat.at[sidx.at[1, 0]], rows[1], gsems[1])
        for j in range(_GRP):
            b = j % 4
            if j + 2 < _GRP:
                nb = (j + 2) % 4
                if j >= 2:
                    sd[j - 2].wait()
                gd[j + 2] = pltpu.async_copy(hflat.at[sidx.at[j + 2, 0]],
                                             rows[nb], gsems[nb])
            gd[j].wait()
            sd[j] = pltpu.async_copy(rows[b], accum.at[didx.at[j, 0]],
                                     ssems[b], add=True)
        for j in range(_GRP - 4, _GRP):
            sd[j].wait()

    def pair(gg, carry):
        fire_idx(2 * gg + 1, sidx_b, didx_b, isem_b)
        drain_idx(sidx_a, didx_a, isem_a)
        process(sidx_a, didx_a)

        @pl.when(gg < _GPAIR - 1)
        def _():
            fire_idx(2 * gg + 2, sidx_a, didx_a, isem_a)

        drain_idx(sidx_b, didx_b, isem_b)
        process(sidx_b, didx_b)
        return carry

    lax.fori_loop(0, _GPAIR, pair, 0)

    plsc.subcore_barrier()
    for k in range(16):
        t = s + _SUB * k

        @pl.when(t < _ZBLK)
        def _():
            pltpu.sync_copy(accum.at[pl.ds(t * _ZROWS, _ZROWS)],
                            out.at[pl.ds(c_n + t * _ZROWS, _ZROWS)])


@functools.lru_cache(maxsize=1)
def _make_edge_agg():
    return pl.kernel(
        _edge_body,
        out_type=jax.ShapeDtypeStruct((2 * _N, _HH), jnp.float32),
        mesh=plsc.VectorSubcoreMesh(core_axis_name="c", subcore_axis_name="s"),
        compiler_params=pltpu.CompilerParams(use_tc_tiling_on_sc=False),
        scratch_types=[
            pltpu.VMEM((_GRP, 1, _CH), jnp.int32),
            pltpu.VMEM((_GRP, 1, _CH), jnp.int32),
            pltpu.VMEM((_GRP, 1, _CH), jnp.int32),
            pltpu.VMEM((_GRP, 1, _CH), jnp.int32),
            pltpu.VMEM((_CH, _HH), jnp.float32),
            pltpu.VMEM((_CH, _HH), jnp.float32),
            pltpu.VMEM((_CH, _HH), jnp.float32),
            pltpu.VMEM((_CH, _HH), jnp.float32),
            pltpu.VMEM((_ZROWS, _HH), jnp.float32),
            pltpu.VMEM_SHARED((_N, _HH), jnp.float32),
            pltpu.SemaphoreType.DMA,
            pltpu.SemaphoreType.DMA,
            pltpu.SemaphoreType.DMA,
            pltpu.SemaphoreType.DMA,
            pltpu.SemaphoreType.DMA,
            pltpu.SemaphoreType.DMA,
            pltpu.SemaphoreType.DMA,
            pltpu.SemaphoreType.DMA,
            pltpu.SemaphoreType.DMA,
            pltpu.SemaphoreType.DMA,
        ],
    )


def _edge_agg(hflat, srcb, dstb):
    return _make_edge_agg()(hflat, srcb, dstb)


def kernel(x, edge_index, batch, coeffs_t, node_W, node_b,
           g1_W1, g1_b1, g1_W2, g1_b2, g1_eps,
           g2_W1, g2_b1, g2_W2, g2_b2, g2_eps,
           g3_W1, g3_b1, g3_W2, g3_b2, g3_eps,
           fc1_W, fc1_b, fc2_W, fc2_b):
    src = edge_index[0].reshape(_NCHUNK, 1, _CH)
    dst = edge_index[1].reshape(_NCHUNK, 1, _CH)
    srcb = jnp.concatenate([src, src + _N], axis=0)
    ids_row = batch.reshape(_NBLK3, 1, _BR3)
    ids_col = batch.reshape(_NBLK3, _BR3, 1)

    hp = _node_lin(x, node_W, node_b, coeffs_t)
    agg = _edge_agg(hp.reshape(2 * _N, _HH), srcb, dst)
    hp = _gin_dense(hp, agg.reshape(2, _N, _HH),
                    g1_W1, g1_b1, g1_W2, g1_b2, g1_eps, coeffs_t)
    agg = _edge_agg(hp.reshape(2 * _N, _HH), srcb, dst)
    hp = _gin_dense(hp, agg.reshape(2, _N, _HH),
                    g2_W1, g2_b1, g2_W2, g2_b2, g2_eps, coeffs_t)
    agg = _edge_agg(hp.reshape(2 * _N, _HH), srcb, dst)
    return _gin3_pool_head(hp, agg.reshape(2, _N, _HH),
                           g3_W1, g3_b1, g3_W2, g3_b2, g3_eps, coeffs_t,
                           ids_row, ids_col, fc1_W, fc1_b, fc2_W, fc2_b)


# speedup vs baseline: 9.4802x; 1.0672x over previous
import functools

import jax
import jax.numpy as jnp
from jax import lax
from jax.experimental import pallas as pl
from jax.experimental.pallas import tpu as pltpu
from jax.experimental.pallas import tpu_sc as plsc

_N = 50000
_F = 128
_H = 64
_HH = 32
_NG = 256
_NCLS = 10
_E = 800000

_SUB = 16
_CH = 125
_GRP = 10
_NCHUNK = _E // _CH
_CPS = _NCHUNK // _SUB
_GPS = _CPS // _GRP
_GPAIR = _GPS // 2
_ZROWS = 200
_ZBLK = _N // _ZROWS

_BR = 2000
_NBLK = _N // _BR
_BR3 = 1000
_NBLK3 = _N // _BR3


def _bf(v):
    return v.astype(jnp.bfloat16).astype(jnp.float32)


def _interp3(cr, w_ref):
    return (_bf(cr[0]) * _bf(w_ref[0]) + _bf(cr[1]) * _bf(w_ref[1])
            + _bf(cr[2]) * _bf(w_ref[2]))


def _mm(a, b):
    return lax.dot_general(a.astype(jnp.bfloat16), b.astype(jnp.bfloat16),
                           (((1,), (1,)), ((), ())),
                           preferred_element_type=jnp.float32)


def _node_body(cr, x_ref, w_ref, b_ref, out_ref):
    wt = _interp3(cr, w_ref)
    bt = _interp3(cr, b_ref)
    h = _mm(x_ref[...], wt) + bt[None, :]
    out_ref[0] = h[:, :_HH]
    out_ref[1] = h[:, _HH:]


def _node_lin(x, node_w, node_b, coeffs):
    return pl.pallas_call(
        _node_body,
        grid=(_NBLK,),
        in_specs=[
            pl.BlockSpec(memory_space=pltpu.SMEM),
            pl.BlockSpec((_BR, _F), lambda r: (r, 0)),
            pl.BlockSpec((3, _H, _F), lambda r: (0, 0, 0)),
            pl.BlockSpec((3, _H), lambda r: (0, 0)),
        ],
        out_specs=pl.BlockSpec((2, _BR, _HH), lambda r: (0, r, 0)),
        out_shape=jax.ShapeDtypeStruct((2, _N, _HH), jnp.float32),
    )(coeffs, x, node_w, node_b)


def _gin_common(cr, er, h_ref, aa_ref, ab_ref, w1_ref, b1_ref, w2_ref,
                b2_ref):
    eps_t = (_bf(cr[0]) * _bf(er[0]) + _bf(cr[1]) * _bf(er[1])
             + _bf(cr[2]) * _bf(er[2]))
    w1t = _interp3(cr, w1_ref)
    b1t = _interp3(cr, b1_ref)
    w2t = _interp3(cr, w2_ref)
    b2t = _interp3(cr, b2_ref)
    h = jnp.concatenate([h_ref[0], h_ref[1]], axis=1)
    agg = jnp.concatenate([aa_ref[:, :_HH], ab_ref[:, :_HH]], axis=1)
    u = (1.0 + eps_t) * h + agg
    m = jnp.maximum(_mm(u, w1t) + b1t[None, :], 0.0)
    o = _mm(m, w2t) + b2t[None, :]
    return jnp.where(o > 0.0, o, jnp.exp(o) - 1.0)


def _gin_body(cr, er, h_ref, aa_ref, ab_ref, w1_ref, b1_ref, w2_ref, b2_ref,
              out_ref):
    o = _gin_common(cr, er, h_ref, aa_ref, ab_ref, w1_ref, b1_ref, w2_ref,
                    b2_ref)
    out_ref[0] = o[:, :_HH]
    out_ref[1] = o[:, _HH:]


def _gin_dense(hp, ap, w1, b1, w2, b2, eps, coeffs):
    return pl.pallas_call(
        _gin_body,
        grid=(_NBLK,),
        in_specs=[
            pl.BlockSpec(memory_space=pltpu.SMEM),
            pl.BlockSpec(memory_space=pltpu.SMEM),
            pl.BlockSpec((2, _BR, _HH), lambda r: (0, r, 0)),
            pl.BlockSpec((_BR, 128), lambda r: (r, 0)),
            pl.BlockSpec((_BR, 128), lambda r: (_NBLK + r, 0)),
            pl.BlockSpec((3, 2 * _H, _H), lambda r: (0, 0, 0)),
            pl.BlockSpec((3, 2 * _H), lambda r: (0, 0)),
            pl.BlockSpec((3, _H, 2 * _H), lambda r: (0, 0, 0)),
            pl.BlockSpec((3, _H), lambda r: (0, 0)),
        ],
        out_specs=pl.BlockSpec((2, _BR, _HH), lambda r: (0, r, 0)),
        out_shape=jax.ShapeDtypeStruct((2, _N, _HH), jnp.float32),
    )(coeffs, eps, hp, ap, ap, w1, b1, w2, b2)


def _gin3_body(cr, er, h_ref, aa_ref, ab_ref, w1_ref, b1_ref, w2_ref, b2_ref,
               idr_ref, idc_ref, fc1w_ref, fc1b_ref, fc2w_ref, fc2b_ref,
               psum_ref, pmax_ref, out_ref):
    r = pl.program_id(0)
    o = _gin_common(cr, er, h_ref, aa_ref, ab_ref, w1_ref, b1_ref, w2_ref,
                    b2_ref)
    ids_row = idr_ref[0]
    ids_col = idc_ref[0]

    iota_g = lax.broadcasted_iota(jnp.int32, (_NG, _BR3), 0)
    oh = (ids_row == iota_g).astype(jnp.float32)
    o_aug = jnp.concatenate([o, jnp.ones((_BR3, _H), jnp.float32)], axis=1)
    psum_blk = lax.dot_general(oh, o_aug, (((1,), (0,)), ((), ())),
                               precision=lax.Precision.HIGHEST,
                               preferred_element_type=jnp.float32)

    g_lo = ids_row[0, 0]
    g_hi = ids_row[0, _BR3 - 1]
    iota_n = lax.broadcasted_iota(jnp.int32, (_NG, 1), 0)
    neg_inf = jnp.float32(-jnp.inf)

    def wbody(carry):
        g, acc = carry
        mx = jnp.max(jnp.where(ids_col == g, o, neg_inf), axis=0)
        acc = jnp.where(iota_n == g, jnp.maximum(acc, mx[None, :]), acc)
        return g + 1, acc

    _, pmax_blk = lax.while_loop(lambda c: c[0] <= g_hi, wbody,
                                 (g_lo, jnp.full((_NG, _H), neg_inf)))

    @pl.when(r == 0)
    def _():
        psum_ref[...] = psum_blk
        pmax_ref[...] = pmax_blk

    @pl.when(r > 0)
    def _():
        psum_ref[...] = psum_ref[...] + psum_blk
        pmax_ref[...] = jnp.maximum(pmax_ref[...], pmax_blk)

    @pl.when(r == _NBLK3 - 1)
    def _():
        ps = psum_ref[...]
        mean = ps[:, :_H] / jnp.maximum(ps[:, _H:], 1.0)
        mx = pmax_ref[...]
        mx = jnp.where(jnp.isfinite(mx), mx, 0.0)
        z = jnp.concatenate([mean, mx], axis=1)
        fc1t = _interp3(cr, fc1w_ref)
        fc1bt = _interp3(cr, fc1b_ref)
        fc2t = _interp3(cr, fc2w_ref)
        fc2bt = _interp3(cr, fc2b_ref)
        z1 = jnp.maximum(_mm(z, fc1t) + fc1bt[None, :], 0.0)
        out_ref[...] = _mm(z1, fc2t) + fc2bt[None, :]


def _gin3_pool_head(hp, ap, w1, b1, w2, b2, eps, coeffs, ids_row, ids_col,
                    fc1w, fc1b, fc2w, fc2b):
    outs = pl.pallas_call(
        _gin3_body,
        grid=(_NBLK3,),
        in_specs=[
            pl.BlockSpec(memory_space=pltpu.SMEM),
            pl.BlockSpec(memory_space=pltpu.SMEM),
            pl.BlockSpec((2, _BR3, _HH), lambda r: (0, r, 0)),
            pl.BlockSpec((_BR3, 128), lambda r: (r, 0)),
            pl.BlockSpec((_BR3, 128), lambda r: (_NBLK3 + r, 0)),
            pl.BlockSpec((3, 2 * _H, _H), lambda r: (0, 0, 0)),
            pl.BlockSpec((3, 2 * _H), lambda r: (0, 0)),
            pl.BlockSpec((3, _H, 2 * _H), lambda r: (0, 0, 0)),
            pl.BlockSpec((3, _H), lambda r: (0, 0)),
            pl.BlockSpec((1, 1, _BR3), lambda r: (r, 0, 0)),
            pl.BlockSpec((1, _BR3, 1), lambda r: (r, 0, 0)),
            pl.BlockSpec((3, _H, 2 * _H), lambda r: (0, 0, 0)),
            pl.BlockSpec((3, _H), lambda r: (0, 0)),
            pl.BlockSpec((3, _NCLS, _H), lambda r: (0, 0, 0)),
            pl.BlockSpec((3, _NCLS), lambda r: (0, 0)),
        ],
        out_specs=[
            pl.BlockSpec((_NG, 2 * _H), lambda r: (0, 0)),
            pl.BlockSpec((_NG, _H), lambda r: (0, 0)),
            pl.BlockSpec((_NG, _NCLS), lambda r: (0, 0)),
        ],
        out_shape=[
            jax.ShapeDtypeStruct((_NG, 2 * _H), jnp.float32),
            jax.ShapeDtypeStruct((_NG, _H), jnp.float32),
            jax.ShapeDtypeStruct((_NG, _NCLS), jnp.float32),
        ],
    )(coeffs, eps, hp, ap, ap, w1, b1, w2, b2, ids_row, ids_col,
      fc1w, fc1b, fc2w, fc2b)
    return outs[2]


def _edge_body(hflat, srcb, dstb, out,
               sidx_a, didx_a, sidx_b, didx_b,
               rows0, rows1, rows2, rows3, zbuf, accum,
               gsem0, gsem1, gsem2, gsem3,
               ssem0, ssem1, ssem2, ssem3, isem_a, isem_b):
    c = lax.axis_index("c")
    s = lax.axis_index("s")
    c_n = c * _N
    rows = (rows0, rows1, rows2, rows3)
    gsems = (gsem0, gsem1, gsem2, gsem3)
    ssems = (ssem0, ssem1, ssem2, ssem3)
    base = s * _CPS

    def fire_idx(g, sidx, didx, isem):
        g0 = base + g * _GRP
        pltpu.async_copy(srcb.at[pl.ds(c * _NCHUNK + g0, _GRP)], sidx, isem)
        pltpu.async_copy(dstb.at[pl.ds(g0, _GRP)], didx, isem)

    def drain_idx(sidx, didx, isem):
        pltpu.make_async_copy(srcb.at[pl.ds(0, _GRP)], sidx, isem).wait()
        pltpu.make_async_copy(dstb.at[pl.ds(0, _GRP)], didx, isem).wait()

    fire_idx(0, sidx_a, didx_a, isem_a)

    zero16 = jnp.zeros((16,), jnp.float32)

    def zb(j, carry):
        zbuf[j, pl.ds(0, 16)] = zero16
        zbuf[j, pl.ds(16, 16)] = zero16
        return carry

    lax.fori_loop(0, _ZROWS, zb, 0)
    for k in range(16):
        t = s + _SUB * k

        @pl.when(t < _ZBLK)
        def _():
            pltpu.sync_copy(zbuf, accum.at[pl.ds(t * _ZROWS, _ZROWS)])

    plsc.subcore_barrier()

    def process(sidx, didx):
        gd = [None] * _GRP
        sd = [None] * _GRP
        gd[0] = pltpu.async_copy(hflat.at[sidx.at[0, 0]], rows[0], gsems[0])
        gd[1] = pltpu.async_copy(hflat.at[sidx.at[1, 0]], rows[1], gsems[1])
        for j in range(_GRP):
            b = j % 4
            if j + 2 < _GRP:
                nb = (j + 2) % 4
                if j >= 2:
                    sd[j - 2].wait()
                gd[j + 2] = pltpu.async_copy(hflat.at[sidx.at[j + 2, 0]],
                                             rows[nb], gsems[nb])
            gd[j].wait()
            sd[j] = pltpu.async_copy(rows[b], accum.at[didx.at[j, 0]],
                                     ssems[b], add=True)
        for j in range(_GRP - 4, _GRP):
            sd[j].wait()

    def pair(gg, carry):
        fire_idx(2 * gg + 1, sidx_b, didx_b, isem_b)
        drain_idx(sidx_a, didx_a, isem_a)
        process(sidx_a, didx_a)

        @pl.when(gg < _GPAIR - 1)
        def _():
            fire_idx(2 * gg + 2, sidx_a, didx_a, isem_a)

        drain_idx(sidx_b, didx_b, isem_b)
        process(sidx_b, didx_b)
        return carry

    lax.fori_loop(0, _GPAIR, pair, 0)

    plsc.subcore_barrier()
    for k in range(16):
        t = s + _SUB * k

        @pl.when(t < _ZBLK)
        def _():
            pltpu.sync_copy(accum.at[pl.ds(t * _ZROWS, _ZROWS)],
                            out.at[pl.ds(c_n + t * _ZROWS, _ZROWS),
                                   pl.ds(0, _HH)])


@functools.lru_cache(maxsize=1)
def _make_edge_agg():
    return pl.kernel(
        _edge_body,
        out_type=jax.ShapeDtypeStruct((2 * _N, 128), jnp.float32),
        mesh=plsc.VectorSubcoreMesh(core_axis_name="c", subcore_axis_name="s"),
        compiler_params=pltpu.CompilerParams(use_tc_tiling_on_sc=False),
        scratch_types=[
            pltpu.VMEM((_GRP, 1, _CH), jnp.int32),
            pltpu.VMEM((_GRP, 1, _CH), jnp.int32),
            pltpu.VMEM((_GRP, 1, _CH), jnp.int32),
            pltpu.VMEM((_GRP, 1, _CH), jnp.int32),
            pltpu.VMEM((_CH, _HH), jnp.float32),
            pltpu.VMEM((_CH, _HH), jnp.float32),
            pltpu.VMEM((_CH, _HH), jnp.float32),
            pltpu.VMEM((_CH, _HH), jnp.float32),
            pltpu.VMEM((_ZROWS, _HH), jnp.float32),
            pltpu.VMEM_SHARED((_N, _HH), jnp.float32),
            pltpu.SemaphoreType.DMA,
            pltpu.SemaphoreType.DMA,
            pltpu.SemaphoreType.DMA,
            pltpu.SemaphoreType.DMA,
            pltpu.SemaphoreType.DMA,
            pltpu.SemaphoreType.DMA,
            pltpu.SemaphoreType.DMA,
            pltpu.SemaphoreType.DMA,
            pltpu.SemaphoreType.DMA,
            pltpu.SemaphoreType.DMA,
        ],
    )


def _edge_agg(hflat, srcb, dstb):
    return _make_edge_agg()(hflat, srcb, dstb)


def kernel(x, edge_index, batch, coeffs_t, node_W, node_b,
           g1_W1, g1_b1, g1_W2, g1_b2, g1_eps,
           g2_W1, g2_b1, g2_W2, g2_b2, g2_eps,
           g3_W1, g3_b1, g3_W2, g3_b2, g3_eps,
           fc1_W, fc1_b, fc2_W, fc2_b):
    src = edge_index[0].reshape(_NCHUNK, 1, _CH)
    dst = edge_index[1].reshape(_NCHUNK, 1, _CH)
    srcb = jnp.concatenate([src, src + _N], axis=0)
    ids_row = batch.reshape(_NBLK3, 1, _BR3)
    ids_col = batch.reshape(_NBLK3, _BR3, 1)

    hp = _node_lin(x, node_W, node_b, coeffs_t)
    agg = _edge_agg(hp.reshape(2 * _N, _HH), srcb, dst)
    hp = _gin_dense(hp, agg, g1_W1, g1_b1, g1_W2, g1_b2, g1_eps, coeffs_t)
    agg = _edge_agg(hp.reshape(2 * _N, _HH), srcb, dst)
    hp = _gin_dense(hp, agg, g2_W1, g2_b1, g2_W2, g2_b2, g2_eps, coeffs_t)
    agg = _edge_agg(hp.reshape(2 * _N, _HH), srcb, dst)
    return _gin3_pool_head(hp, agg,
                           g3_W1, g3_b1, g3_W2, g3_b2, g3_eps, coeffs_t,
                           ids_row, ids_col, fc1_W, fc1_b, fc2_W, fc2_b)
